# Initial kernel scaffold; baseline (speedup 1.0000x reference)
import jax
import jax.numpy as jnp
from jax.experimental import pallas as pl


def kernel(feat, edge_index, W1, b1, W2, b2, eps):
    raise NotImplementedError("write your pallas kernel here")


# speedup vs baseline: 5.3646x; 5.3646x over previous
import functools

import jax
import jax.numpy as jnp
from jax import lax
from jax.experimental import pallas as pl
from jax.experimental.pallas import tpu as pltpu
from jax.experimental.pallas import tpu_sc as plsc

N_NODES = 10000
N_EDGES = 320000
D_FEAT = 128

NC = 2
NS = 16
NW = NC * NS

EPW = N_EDGES // NW
CHUNK = 80
N_CHUNKS = EPW // CHUNK
N_PAD = 10240
ROWS_PER_TILE = N_PAD // NS


def _sc_segment_sum(feat, src, dst, zeros_tile):
    mesh = plsc.VectorSubcoreMesh(core_axis_name="c", subcore_axis_name="s")

    @functools.partial(
        pl.kernel,
        out_type=jax.ShapeDtypeStruct((2 * N_PAD, D_FEAT), jnp.float32),
        mesh=mesh,
        scratch_types=[
            pltpu.VMEM((CHUNK,), jnp.int32),
            pltpu.VMEM((CHUNK,), jnp.int32),
            pltpu.VMEM((CHUNK, D_FEAT), jnp.float32),
            pltpu.VMEM_SHARED((N_PAD, D_FEAT), jnp.float32),
            pltpu.SemaphoreType.DMA,
        ],
    )
    def k(feat_hbm, src_hbm, dst_hbm, zeros_hbm, out_hbm,
          src_v, dst_v, rows_v, acc_sh, sem):
        cid = lax.axis_index("c")
        sid = lax.axis_index("s")
        wid = sid * NC + cid

        pltpu.sync_copy(zeros_hbm, acc_sh.at[pl.ds(sid * ROWS_PER_TILE, ROWS_PER_TILE)])
        plsc.subcore_barrier()

        base = wid * EPW

        def body(c, carry):
            off = base + c * CHUNK
            pltpu.sync_copy(src_hbm.at[pl.ds(off, CHUNK)], src_v)
            pltpu.sync_copy(dst_hbm.at[pl.ds(off, CHUNK)], dst_v)
            pltpu.async_copy(feat_hbm.at[src_v], rows_v, sem).wait()
            pltpu.sync_copy(rows_v, acc_sh.at[dst_v], add=True)
            return carry

        lax.fori_loop(0, N_CHUNKS, body, 0)

        plsc.subcore_barrier()
        row0 = sid * ROWS_PER_TILE
        pltpu.sync_copy(
            acc_sh.at[pl.ds(row0, ROWS_PER_TILE)],
            out_hbm.at[pl.ds(cid * N_PAD + row0, ROWS_PER_TILE)],
        )

    out = k(feat, src, dst, zeros_tile)
    return out.reshape(2, N_PAD, D_FEAT)[:, :N_NODES, :]


ROW_BLK = 1000


def _mlp_body(scale_ref, feat_ref, p0_ref, p1_ref, w1_ref, b1_ref,
              w2_ref, b2_ref, out_ref):
    h = scale_ref[0] * feat_ref[...] + p0_ref[...] + p1_ref[...]
    h1 = jnp.dot(h, w1_ref[...], preferred_element_type=jnp.float32) + b1_ref[...]
    h1 = jnp.maximum(h1, 0.0)
    out_ref[...] = (
        jnp.dot(h1, w2_ref[...], preferred_element_type=jnp.float32) + b2_ref[...]
    )


def _tc_mlp(feat, p0, p1, W1, b1, W2, b2, scale):
    n_blocks = N_NODES // ROW_BLK
    grid = (n_blocks,)
    row_spec = pl.BlockSpec((ROW_BLK, D_FEAT), lambda i: (i, 0))
    full_spec = pl.BlockSpec((D_FEAT, D_FEAT), lambda i: (0, 0))
    bias_spec = pl.BlockSpec((1, D_FEAT), lambda i: (0, 0))
    return pl.pallas_call(
        _mlp_body,
        grid=grid,
        in_specs=[
            pl.BlockSpec(memory_space=pltpu.SMEM),
            row_spec, row_spec, row_spec,
            full_spec, bias_spec, full_spec, bias_spec,
        ],
        out_specs=row_spec,
        out_shape=jax.ShapeDtypeStruct((N_NODES, D_FEAT), jnp.float32),
    )(scale, feat, p0, p1, W1, b1.reshape(1, D_FEAT), W2, b2.reshape(1, D_FEAT))


@jax.jit
def kernel(feat, edge_index, W1, b1, W2, b2, eps):
    src = edge_index[0].astype(jnp.int32)
    dst = edge_index[1].astype(jnp.int32)
    zeros_tile = jnp.zeros((ROWS_PER_TILE, D_FEAT), jnp.float32)
    partials = _sc_segment_sum(feat, src, dst, zeros_tile)
    scale = jnp.reshape(1.0 + eps, (1,)).astype(jnp.float32)
    return _tc_mlp(feat, partials[0], partials[1], W1, b1, W2, b2, scale)


# speedup vs baseline: 11.9348x; 2.2247x over previous
import functools

import jax
import jax.numpy as jnp
from jax import lax
from jax.experimental import pallas as pl
from jax.experimental.pallas import tpu as pltpu
from jax.experimental.pallas import tpu_sc as plsc

N_NODES = 10000
N_EDGES = 320000
D_FEAT = 128

NC = 2
NS = 16
NW = NC * NS

EPW = N_EDGES // NW
CHUNK = 80
N_CHUNKS = EPW // CHUNK
N_PAIRS = (N_CHUNKS - 1) // 2
N_PAD = 10240
ROWS_PER_TILE = N_PAD // NS


def _sc_segment_sum(feat, src, dst, zeros_tile):
    mesh = plsc.VectorSubcoreMesh(core_axis_name="c", subcore_axis_name="s")

    @functools.partial(
        pl.kernel,
        out_type=jax.ShapeDtypeStruct((2, N_PAD, D_FEAT), jnp.float32),
        mesh=mesh,
        scratch_types=[
            pltpu.VMEM((EPW,), jnp.int32),
            pltpu.VMEM((CHUNK,), jnp.int32),
            pltpu.VMEM((CHUNK,), jnp.int32),
            pltpu.VMEM((CHUNK, D_FEAT), jnp.float32),
            pltpu.VMEM((CHUNK, D_FEAT), jnp.float32),
            pltpu.VMEM_SHARED((N_PAD, D_FEAT), jnp.float32),
            pltpu.SemaphoreType.DMA,
            pltpu.SemaphoreType.DMA,
            pltpu.SemaphoreType.DMA,
            pltpu.SemaphoreType.DMA,
        ],
    )
    def k(feat_hbm, src_hbm, dst_hbm, zeros_hbm, out_hbm,
          src_v, dstb0, dstb1, rows0, rows1, acc_sh, semg0, semg1, semi0, semi1):
        cid = lax.axis_index("c")
        sid = lax.axis_index("s")
        wid = sid * NC + cid
        base = wid * EPW

        pltpu.sync_copy(src_hbm.at[pl.ds(base, EPW)], src_v)
        pltpu.sync_copy(zeros_hbm, acc_sh.at[pl.ds(sid * ROWS_PER_TILE, ROWS_PER_TILE)])
        plsc.subcore_barrier()

        def start_chunk(c, dstb, rows, semi, semg):
            pltpu.async_copy(dst_hbm.at[pl.ds(base + c * CHUNK, CHUNK)], dstb, semi)
            pltpu.async_copy(feat_hbm.at[src_v.at[pl.ds(c * CHUNK, CHUNK)]], rows, semg)

        def drain_chunk(dstb, rows, semi, semg):
            pltpu.make_async_copy(dst_hbm.at[pl.ds(0, CHUNK)], dstb, semi).wait()
            pltpu.make_async_copy(feat_hbm.at[pl.ds(0, CHUNK)], rows, semg).wait()

        def scatter(dstb, rows):
            pltpu.sync_copy(rows, acc_sh.at[dstb], add=True)

        start_chunk(0, dstb0, rows0, semi0, semg0)

        def pair(i, carry):
            c0 = 2 * i
            start_chunk(c0 + 1, dstb1, rows1, semi1, semg1)
            drain_chunk(dstb0, rows0, semi0, semg0)
            scatter(dstb0, rows0)
            start_chunk(c0 + 2, dstb0, rows0, semi0, semg0)
            drain_chunk(dstb1, rows1, semi1, semg1)
            scatter(dstb1, rows1)
            return carry

        lax.fori_loop(0, N_PAIRS, pair, 0)

        drain_chunk(dstb0, rows0, semi0, semg0)
        scatter(dstb0, rows0)

        plsc.subcore_barrier()
        row0 = sid * ROWS_PER_TILE
        pltpu.sync_copy(
            acc_sh.at[pl.ds(row0, ROWS_PER_TILE)],
            out_hbm.at[cid, pl.ds(row0, ROWS_PER_TILE)],
        )

    return k(feat, src, dst, zeros_tile)


ROW_BLK = 1000


def _mlp_body(scale_ref, feat_ref, p_ref, w1_ref, b1_ref, w2_ref, b2_ref, out_ref):
    h = scale_ref[0] * feat_ref[...] + p_ref[0] + p_ref[1]
    h1 = jnp.dot(h, w1_ref[...], preferred_element_type=jnp.float32) + b1_ref[...]
    h1 = jnp.maximum(h1, 0.0)
    out_ref[...] = (
        jnp.dot(h1, w2_ref[...], preferred_element_type=jnp.float32) + b2_ref[...]
    )


def _tc_mlp(feat, partials, W1, b1, W2, b2, scale):
    n_blocks = N_NODES // ROW_BLK
    row_spec = pl.BlockSpec((ROW_BLK, D_FEAT), lambda i: (i, 0))
    p_spec = pl.BlockSpec((2, ROW_BLK, D_FEAT), lambda i: (0, i, 0))
    full_spec = pl.BlockSpec((D_FEAT, D_FEAT), lambda i: (0, 0))
    bias_spec = pl.BlockSpec((1, D_FEAT), lambda i: (0, 0))
    return pl.pallas_call(
        _mlp_body,
        grid=(n_blocks,),
        in_specs=[
            pl.BlockSpec(memory_space=pltpu.SMEM),
            row_spec, p_spec, full_spec, bias_spec, full_spec, bias_spec,
        ],
        out_specs=row_spec,
        out_shape=jax.ShapeDtypeStruct((N_NODES, D_FEAT), jnp.float32),
    )(scale, feat, partials, W1, b1.reshape(1, D_FEAT), W2, b2.reshape(1, D_FEAT))


@jax.jit
def kernel(feat, edge_index, W1, b1, W2, b2, eps):
    src = edge_index[0].astype(jnp.int32)
    dst = edge_index[1].astype(jnp.int32)
    zeros_tile = jnp.zeros((ROWS_PER_TILE, D_FEAT), jnp.float32)
    partials = _sc_segment_sum(feat, src, dst, zeros_tile)
    scale = jnp.reshape(1.0 + eps, (1,)).astype(jnp.float32)
    return _tc_mlp(feat, partials, W1, b1, W2, b2, scale)
